# baseline (speedup 1.0000x reference)
import functools

import jax
import jax.numpy as jnp
from jax.experimental import pallas as pl
from jax.experimental.pallas import tpu as pltpu

D_MODEL = 1024
BATCH = 16384

_K = 16
_NSTEP = BATCH // _K


def _tc_body(idx_smem, pe_any, out_any, sem):
    def step(i, carry):
        base = i * _K
        for k in range(_K):
            r = idx_smem[base + k]
            pltpu.make_async_copy(pe_any.at[r], out_any.at[base + k],
                                  sem).start()
        for k in range(_K):
            pltpu.make_async_copy(pe_any.at[0], out_any.at[0], sem).wait()
        return carry

    jax.lax.fori_loop(0, _NSTEP, step, 0)


@jax.jit
def tc_gather(pe, index):
    grid_spec = pltpu.PrefetchScalarGridSpec(
        num_scalar_prefetch=1,
        grid=(1,),
        in_specs=[pl.BlockSpec(memory_space=pl.ANY)],
        out_specs=pl.BlockSpec(memory_space=pl.ANY),
        scratch_shapes=[pltpu.SemaphoreType.DMA],
    )
    return pl.pallas_call(
        _tc_body,
        grid_spec=grid_spec,
        out_shape=jax.ShapeDtypeStruct((BATCH, D_MODEL), jnp.float32),
    )(index.astype(jnp.int32), pe)


def kernel(pe, index):
    return tc_gather(pe, index)


# speedup vs baseline: 47.3201x; 47.3201x over previous
import functools

import jax
import jax.numpy as jnp
from jax import lax
from jax.experimental import pallas as pl
from jax.experimental.pallas import tpu as pltpu
from jax.experimental.pallas import tpu_sc as plsc

D_MODEL = 1024
MAX_LEN = 8192
BATCH = 16384

_NC = 2
_NS = 16
_NW = _NC * _NS

_B_PER_W = BATCH // _NW
_C = 32
_NCH = _B_PER_W // _C
_NBUF = 3


def _make_gather():
    mesh = plsc.VectorSubcoreMesh(core_axis_name="c", subcore_axis_name="s")

    @functools.partial(
        pl.kernel,
        mesh=mesh,
        out_type=jax.ShapeDtypeStruct((BATCH, D_MODEL), jnp.float32),
        scratch_types=[
            pltpu.VMEM((_B_PER_W,), jnp.int32),
            pltpu.VMEM((_NBUF, _C, D_MODEL), jnp.float32),
        ] + [pltpu.SemaphoreType.DMA] * (2 * _NBUF),
    )
    def gather_kernel(table_hbm, idx_hbm, out_hbm, idx_v, rows_v, *sems):
        wid = lax.axis_index("s") * _NC + lax.axis_index("c")
        base = wid * _B_PER_W
        pltpu.sync_copy(idx_hbm.at[pl.ds(base, _B_PER_W)], idx_v)

        gsems = sems[:_NBUF]
        ssems = sems[_NBUF:]
        gathers = [None] * _NBUF
        scatters = [None] * _NBUF
        for j in range(_NBUF - 1):
            gathers[j] = pltpu.async_copy(
                table_hbm.at[idx_v.at[pl.ds(j * _C, _C)]], rows_v.at[j], gsems[j])
        for j in range(_NCH):
            b = j % _NBUF
            ahead = j + _NBUF - 1
            nb = ahead % _NBUF
            if ahead < _NCH:
                if scatters[nb] is not None:
                    scatters[nb].wait()
                gathers[nb] = pltpu.async_copy(
                    table_hbm.at[idx_v.at[pl.ds(ahead * _C, _C)]], rows_v.at[nb], gsems[nb])
            gathers[b].wait()
            scatters[b] = pltpu.async_copy(
                rows_v.at[b], out_hbm.at[pl.ds(base + j * _C, _C)], ssems[b])
        for b in range(_NBUF):
            if scatters[b] is not None:
                scatters[b].wait()

    return gather_kernel


_gather = _make_gather()


def kernel(pe, index):
    return _gather(pe, index)
